# baseline (speedup 1.0000x reference)
import jax
import jax.numpy as jnp
from jax.experimental import pallas as pl
from jax.experimental.pallas import tpu as pltpu
from jax.experimental.pallas import tpu_sc as plsc

_REG = 1e-5
_GATHER_WINDOW = 256
_TC_CHUNK = 2048


def _sc_gather(packed_table, idx):
    n = idx.shape[0]
    width = packed_table.shape[1]
    idx2 = idx.reshape(1, n)
    mesh = plsc.VectorSubcoreMesh(core_axis_name="core", subcore_axis_name="subcore")

    @pl.kernel(
        out_type=jax.ShapeDtypeStruct((n, width), packed_table.dtype),
        mesh=mesh,
    )
    def gather_kernel(x_hbm, i_hbm, o_hbm):
        def body(i_vmem, o_vmem):
            pltpu.sync_copy(x_hbm.at[i_vmem.at[0]], o_vmem)

        pltpu.emit_pipeline(
            body,
            grid=(n // _GATHER_WINDOW,),
            in_specs=[pl.BlockSpec((1, _GATHER_WINDOW), index_map=lambda i: (0, i))],
            out_specs=[pl.BlockSpec((_GATHER_WINDOW, width), index_map=lambda i: (i, 0))],
            core_axis_name=("core", "subcore"),
            dimension_semantics=(pltpu.PARALLEL,),
        )(i_hbm, o_hbm)

    return gather_kernel(packed_table, idx2)


def _tc_reduce_partial(gathered, parity):
    width = gathered.shape[2]
    half = width // 2
    n_steps = gathered.shape[1] // _TC_CHUNK

    def body(g_ref, par_ref, out_ref, acc_ref):
        i = pl.program_id(0)

        @pl.when(i == 0)
        def _():
            acc_ref[0] = 0.0
            acc_ref[1] = 0.0

        lane = jax.lax.broadcasted_iota(jnp.int32, (_TC_CHUNK, width), 1)
        lane_lo = lane < half

        def mirror(k):
            par = par_ref[k][:, None] != 0
            m = jnp.where(lane_lo != par, g_ref[k], 0.0)
            return m + pltpu.roll(m, half, 1)

        u = mirror(0)
        p = mirror(1)
        ng = mirror(2)
        d = 0.5 * jnp.sum(u * (p - ng), axis=1)
        acc_ref[0] += jnp.sum(jax.nn.log_sigmoid(d.reshape(-1, 128)))
        acc_ref[1] += 0.5 * (jnp.sum(u * u) + jnp.sum(p * p) + jnp.sum(ng * ng))

        @pl.when(i == n_steps - 1)
        def _():
            out_ref[0, 0] = acc_ref[0]
            out_ref[0, 1] = acc_ref[1]

    return pl.pallas_call(
        body,
        grid=(n_steps,),
        in_specs=[
            pl.BlockSpec((3, _TC_CHUNK, width), lambda i: (0, i, 0)),
            pl.BlockSpec((3, _TC_CHUNK), lambda i: (0, i)),
        ],
        out_shape=jax.ShapeDtypeStruct((1, 2), jnp.float32),
        out_specs=pl.BlockSpec(memory_space=pltpu.SMEM),
        scratch_shapes=[pltpu.SMEM((2,), jnp.float32)],
    )(gathered, parity)


def kernel(all_embed, u_id, pos_i_id, neg_i_id):
    batch = u_id.shape[0]
    n_rows, emb = all_embed.shape
    m = batch // 2
    packed = all_embed.reshape(n_rows // 2, 2 * emb)

    ids = jnp.stack([u_id.astype(jnp.int32), pos_i_id.astype(jnp.int32),
                     neg_i_id.astype(jnp.int32)])
    parts = []
    for s in range(2):
        idx = ids[:, s * m:(s + 1) * m].reshape(-1)
        gathered = _sc_gather(packed, idx // 2).reshape(3, m, 2 * emb)
        parity = (idx & 1).reshape(3, m)
        parts.append(_tc_reduce_partial(gathered, parity))

    total = parts[0][0] + parts[1][0]
    bpr = -total[0] / batch
    emb_loss = _REG * total[1] / (2.0 * batch)
    loss = bpr + emb_loss
    reward = jnp.float32(0.0)
    return (reward, loss, bpr, emb_loss)
